# baseline (speedup 1.0000x reference)
import functools

import jax
import jax.numpy as jnp
from jax import lax
from jax.experimental import pallas as pl
from jax.experimental.pallas import tpu as pltpu
from jax.experimental.pallas import tpu_sc as plsc

NN = 10000
NPAD = 10112
EE = 320000
DD = 128
BB = 64
NTILES = 32
E_PER_TILE = EE // NTILES
CH = 80
NCHUNK = E_PER_TILE // CH
NBUF = 4
NIDX = 8
ROWS_PER_TILE = NPAD // 16
ZCH = ROWS_PER_TILE // 8
RB = 632
NRB = NPAD // RB


def _sc_agg(table, ei4):
    mesh = plsc.VectorSubcoreMesh(core_axis_name="c", subcore_axis_name="s")

    @functools.partial(
        pl.kernel,
        out_type=jax.ShapeDtypeStruct((2 * NPAD, DD), jnp.float32),
        mesh=mesh,
        scratch_types=[
            pltpu.VMEM((NIDX, 2, CH), jnp.int32),
            [pltpu.VMEM((CH, DD), jnp.float32) for _ in range(NBUF)],
            pltpu.VMEM_SHARED((NPAD, DD), jnp.float32),
            [pltpu.SemaphoreType.DMA for _ in range(NIDX)],
            [pltpu.SemaphoreType.DMA for _ in range(NBUF)],
            [pltpu.SemaphoreType.DMA for _ in range(NBUF)],
        ],
    )
    def k(table_h, ei_h, out_h, ring, rows, acc_s, isem, gsem, ssem):
        cid = lax.axis_index("c")
        sid = lax.axis_index("s")
        tile = cid * 16 + sid

        def zrow(r, _):
            def zcol(j, _):
                rows[0][r, pl.ds(j * 16, 16)] = jnp.zeros((16,), jnp.float32)
                return 0
            return lax.fori_loop(0, DD // 16, zcol, 0)
        lax.fori_loop(0, CH, zrow, 0)

        def zacc(i, _):
            pltpu.sync_copy(
                rows[0].at[pl.ds(0, ZCH)],
                acc_s.at[pl.ds(sid * ROWS_PER_TILE + i * ZCH, ZCH)])
            return 0
        lax.fori_loop(0, ROWS_PER_TILE // ZCH, zacc, 0)
        plsc.subcore_barrier()

        def iload(c, s):
            pltpu.async_copy(ei_h.at[tile, c], ring.at[s], isem[s])

        def iwait(c, s):
            pltpu.make_async_copy(ei_h.at[tile, c], ring.at[s],
                                  isem[s]).wait()

        HC = CH // 2

        def gstart(s, b):
            pltpu.async_copy(table_h.at[ring.at[s, 0, pl.ds(0, HC)]],
                             rows[b].at[pl.ds(0, HC)], gsem[b])
            pltpu.async_copy(table_h.at[ring.at[s, 0, pl.ds(HC, HC)]],
                             rows[b].at[pl.ds(HC, HC)], gsem[b])

        def gwait(s, b):
            pltpu.make_async_copy(table_h.at[ring.at[s, 0, pl.ds(0, HC)]],
                                  rows[b].at[pl.ds(0, HC)], gsem[b]).wait()
            pltpu.make_async_copy(table_h.at[ring.at[s, 0, pl.ds(HC, HC)]],
                                  rows[b].at[pl.ds(HC, HC)], gsem[b]).wait()

        def sstart(s, b):
            pltpu.async_copy(rows[b], acc_s.at[ring.at[s, 1]], ssem[b],
                             add=True)

        def swait(s, b):
            pltpu.make_async_copy(rows[b], acc_s.at[ring.at[s, 1]],
                                  ssem[b]).wait()

        def iter_ops(j, jm8, jm4, do_swait, do_iload, do_gather):
            gwait(jm8, jm4)
            sstart(jm8, jm4)
            if do_swait:
                swait((jm8 + 7) % NIDX, (jm4 + 3) % NBUF)
            if do_iload:
                iload(j + 6, (jm8 + 6) % NIDX)
            if do_gather:
                iwait(j + 3, (jm8 + 3) % NIDX)
                gstart((jm8 + 3) % NIDX, (jm4 + 3) % NBUF)

        for c in range(6):
            iload(c, c)
        for c in range(3):
            iwait(c, c)
            gstart(c, c)
        iter_ops(0, 0, 0, False, True, True)
        for j in range(1, 10):
            iter_ops(j, j % NIDX, j % NBUF, True, True, True)

        def body(o, _):
            j0 = 10 + o * 8
            for t in range(8):
                iter_ops(j0 + t, (10 + t) % NIDX, (10 + t) % NBUF,
                         True, True, True)
            return 0
        lax.fori_loop(0, 13, body, 0)

        for j in range(114, 119):
            iter_ops(j, j % NIDX, j % NBUF, True, True, True)
        for j in range(119, 122):
            iter_ops(j, j % NIDX, j % NBUF, True, False, True)
        for j in (122, 123, 124):
            jm8, jm4 = j % NIDX, j % NBUF
            gwait(jm8, jm4)
            sstart(jm8, jm4)
            swait((jm8 + 7) % NIDX, (jm4 + 3) % NBUF)
        swait(124 % NIDX, 124 % NBUF)
        plsc.subcore_barrier()

        pltpu.sync_copy(
            acc_s.at[pl.ds(sid * ROWS_PER_TILE, ROWS_PER_TILE)],
            out_h.at[pl.ds(cid * NPAD + sid * ROWS_PER_TILE, ROWS_PER_TILE)])

    return k(table, ei4)


def _mlp0(x, aggs, W1, b1, W2, b2):
    def body(x_r, a0_r, a1_r, w1_r, b1_r, w2_r, b2_r, o_r):
        h = x_r[...] + a0_r[...] + a1_r[...]
        h = jnp.maximum(
            jnp.dot(h, w1_r[...], preferred_element_type=jnp.float32)
            + b1_r[...], 0.0)
        h = jnp.dot(h, w2_r[...], preferred_element_type=jnp.float32) + b2_r[...]
        o_r[...] = jnp.maximum(h, 0.0)

    row = pl.BlockSpec((RB, DD), lambda i: (i, 0))
    row_hi = pl.BlockSpec((RB, DD), lambda i: (i + NRB, 0))
    full = pl.BlockSpec((DD, DD), lambda i: (0, 0))
    bias = pl.BlockSpec((1, DD), lambda i: (0, 0))
    return pl.pallas_call(
        body,
        grid=(NRB,),
        in_specs=[row, row, row_hi, full, bias, full, bias],
        out_specs=row,
        out_shape=jax.ShapeDtypeStruct((NPAD, DD), jnp.float32),
    )(x, aggs, aggs, W1, b1.reshape(1, DD), W2, b2.reshape(1, DD))


def _mlp1_pool(h0, aggs, W1, b1, W2, b2, batch3d):
    def body(h_r, a0_r, a1_r, w1_r, b1_r, w2_r, b2_r, bt_r, o_r, acc, cnt):
        i = pl.program_id(0)

        @pl.when(i == 0)
        def _():
            acc[...] = jnp.zeros_like(acc)
            cnt[...] = jnp.zeros_like(cnt)

        h = h_r[...] + a0_r[...] + a1_r[...]
        h = jnp.maximum(
            jnp.dot(h, w1_r[...], preferred_element_type=jnp.float32)
            + b1_r[...], 0.0)
        h = jnp.dot(h, w2_r[...], preferred_element_type=jnp.float32) + b2_r[...]

        seg = bt_r[...].reshape(1, RB)
        onehot = (jnp.broadcast_to(seg, (BB, RB))
                  == lax.broadcasted_iota(jnp.int32, (BB, RB), 0)
                  ).astype(jnp.float32)
        acc[...] += jnp.dot(onehot, h, preferred_element_type=jnp.float32)
        cnt[...] += jnp.broadcast_to(
            jnp.sum(onehot, axis=1, keepdims=True), (BB, DD))

        @pl.when(i == NRB - 1)
        def _():
            o_r[...] = acc[...] / jnp.maximum(cnt[...], 1.0)

    row = pl.BlockSpec((RB, DD), lambda i: (i, 0))
    row_hi = pl.BlockSpec((RB, DD), lambda i: (i + NRB, 0))
    full = pl.BlockSpec((DD, DD), lambda i: (0, 0))
    bias = pl.BlockSpec((1, DD), lambda i: (0, 0))
    return pl.pallas_call(
        body,
        grid=(NRB,),
        in_specs=[row, row, row_hi, full, bias, full, bias,
                  pl.BlockSpec((1, 1, RB), lambda i: (i, 0, 0))],
        out_specs=pl.BlockSpec((BB, DD), lambda i: (0, 0)),
        out_shape=jax.ShapeDtypeStruct((BB, DD), jnp.float32),
        scratch_shapes=[pltpu.VMEM((BB, DD), jnp.float32),
                        pltpu.VMEM((BB, DD), jnp.float32)],
    )(h0, aggs, aggs, W1, b1.reshape(1, DD), W2, b2.reshape(1, DD), batch3d)


def kernel(x, edge_index, batch, W1_0, b1_0, W2_0, b2_0, W1_1, b1_1, W2_1, b2_1):
    ei4 = edge_index.reshape(2, NTILES, NCHUNK, CH).transpose(1, 2, 0, 3)
    x_pad = jnp.pad(x, ((0, NPAD - NN), (0, 0)))
    batch3d = jnp.pad(batch, (0, NPAD - NN),
                      constant_values=BB).reshape(NRB, 1, RB)

    aggs = _sc_agg(x_pad, ei4)
    h0 = _mlp0(x_pad, aggs, W1_0, b1_0, W2_0, b2_0)
    aggs1 = _sc_agg(h0, ei4)
    return _mlp1_pool(h0, aggs1, W1_1, b1_1, W2_1, b2_1, batch3d)


# speedup vs baseline: 1.0073x; 1.0073x over previous
import functools

import jax
import jax.numpy as jnp
from jax import lax
from jax.experimental import pallas as pl
from jax.experimental.pallas import tpu as pltpu
from jax.experimental.pallas import tpu_sc as plsc

NN = 10000
NPAD = 10112
EE = 320000
DD = 128
BB = 64
NTILES = 32
E_PER_TILE = EE // NTILES
CH = 80
NCHUNK = E_PER_TILE // CH
NBUF = 4
NIDX = 8
ROWS_PER_TILE = NPAD // 16
ZCH = ROWS_PER_TILE // 8
RB = 632
NRB = NPAD // RB


def _sc_agg(table, ei4):
    mesh = plsc.VectorSubcoreMesh(core_axis_name="c", subcore_axis_name="s")

    @functools.partial(
        pl.kernel,
        out_type=jax.ShapeDtypeStruct((2 * NPAD, DD), jnp.float32),
        mesh=mesh,
        scratch_types=[
            pltpu.VMEM((NIDX, 2, CH), jnp.int32),
            [pltpu.VMEM((CH, DD), jnp.float32) for _ in range(NBUF)],
            pltpu.VMEM_SHARED((NPAD, DD), jnp.float32),
            [pltpu.SemaphoreType.DMA for _ in range(NIDX)],
            [pltpu.SemaphoreType.DMA for _ in range(NBUF)],
            [pltpu.SemaphoreType.DMA for _ in range(NBUF)],
        ],
    )
    def k(table_h, ei_h, out_h, ring, rows, acc_s, isem, gsem, ssem):
        cid = lax.axis_index("c")
        sid = lax.axis_index("s")
        tile = cid * 16 + sid

        def zrow(r, _):
            def zcol(j, _):
                rows[0][r, pl.ds(j * 16, 16)] = jnp.zeros((16,), jnp.float32)
                return 0
            return lax.fori_loop(0, DD // 16, zcol, 0)
        lax.fori_loop(0, CH, zrow, 0)

        def zacc(i, _):
            pltpu.sync_copy(
                rows[0].at[pl.ds(0, ZCH)],
                acc_s.at[pl.ds(sid * ROWS_PER_TILE + i * ZCH, ZCH)])
            return 0
        lax.fori_loop(0, ROWS_PER_TILE // ZCH, zacc, 0)
        plsc.subcore_barrier()

        def iload(c, s):
            pltpu.async_copy(ei_h.at[tile, c], ring.at[s], isem[s])

        def iwait(c, s):
            pltpu.make_async_copy(ei_h.at[tile, c], ring.at[s],
                                  isem[s]).wait()

        def gstart(s, b):
            pltpu.async_copy(table_h.at[ring.at[s, 0]], rows[b], gsem[b])

        def gwait(s, b):
            pltpu.make_async_copy(table_h.at[ring.at[s, 0]], rows[b],
                                  gsem[b]).wait()

        def sstart(s, b):
            pltpu.async_copy(rows[b], acc_s.at[ring.at[s, 1]], ssem[b],
                             add=True)

        def swait(s, b):
            pltpu.make_async_copy(rows[b], acc_s.at[ring.at[s, 1]],
                                  ssem[b]).wait()

        def iter_ops(j, jm8, jm4, do_swait, do_iload, do_gather):
            gwait(jm8, jm4)
            sstart(jm8, jm4)
            if do_swait:
                swait((jm8 + 7) % NIDX, (jm4 + 3) % NBUF)
            if do_iload:
                iload(j + 6, (jm8 + 6) % NIDX)
            if do_gather:
                iwait(j + 3, (jm8 + 3) % NIDX)
                gstart((jm8 + 3) % NIDX, (jm4 + 3) % NBUF)

        for c in range(6):
            iload(c, c)
        for c in range(3):
            iwait(c, c)
            gstart(c, c)
        iter_ops(0, 0, 0, False, True, True)
        for j in range(1, 10):
            iter_ops(j, j % NIDX, j % NBUF, True, True, True)

        def body(o, _):
            j0 = 10 + o * 8
            for t in range(8):
                iter_ops(j0 + t, (10 + t) % NIDX, (10 + t) % NBUF,
                         True, True, True)
            return 0
        lax.fori_loop(0, 13, body, 0)

        for j in range(114, 119):
            iter_ops(j, j % NIDX, j % NBUF, True, True, True)
        for j in range(119, 122):
            iter_ops(j, j % NIDX, j % NBUF, True, False, True)
        for j in (122, 123, 124):
            jm8, jm4 = j % NIDX, j % NBUF
            gwait(jm8, jm4)
            sstart(jm8, jm4)
            swait((jm8 + 7) % NIDX, (jm4 + 3) % NBUF)
        swait(124 % NIDX, 124 % NBUF)
        plsc.subcore_barrier()

        pltpu.sync_copy(
            acc_s.at[pl.ds(sid * ROWS_PER_TILE, ROWS_PER_TILE)],
            out_h.at[pl.ds(cid * NPAD + sid * ROWS_PER_TILE, ROWS_PER_TILE)])

    return k(table, ei4)


def _mlp0(x, aggs, W1, b1, W2, b2):
    def body(x_r, a0_r, a1_r, w1_r, b1_r, w2_r, b2_r, o_r):
        h = x_r[...] + a0_r[...] + a1_r[...]
        h = jnp.maximum(
            jnp.dot(h, w1_r[...], preferred_element_type=jnp.float32)
            + b1_r[...], 0.0)
        h = jnp.dot(h, w2_r[...], preferred_element_type=jnp.float32) + b2_r[...]
        o_r[...] = jnp.maximum(h, 0.0)

    row = pl.BlockSpec((RB, DD), lambda i: (i, 0))
    row_hi = pl.BlockSpec((RB, DD), lambda i: (i + NRB, 0))
    full = pl.BlockSpec((DD, DD), lambda i: (0, 0))
    bias = pl.BlockSpec((1, DD), lambda i: (0, 0))
    return pl.pallas_call(
        body,
        grid=(NRB,),
        in_specs=[row, row, row_hi, full, bias, full, bias],
        out_specs=row,
        out_shape=jax.ShapeDtypeStruct((NPAD, DD), jnp.float32),
    )(x, aggs, aggs, W1, b1.reshape(1, DD), W2, b2.reshape(1, DD))


def _mlp1_pool(h0, aggs, W1, b1, W2, b2, batch3d):
    def body(h_r, a0_r, a1_r, w1_r, b1_r, w2_r, b2_r, bt_r, o_r, acc, cnt):
        i = pl.program_id(0)

        @pl.when(i == 0)
        def _():
            acc[...] = jnp.zeros_like(acc)
            cnt[...] = jnp.zeros_like(cnt)

        h = h_r[...] + a0_r[...] + a1_r[...]
        h = jnp.maximum(
            jnp.dot(h, w1_r[...], preferred_element_type=jnp.float32)
            + b1_r[...], 0.0)
        h = jnp.dot(h, w2_r[...], preferred_element_type=jnp.float32) + b2_r[...]

        seg = bt_r[...].reshape(1, RB)
        onehot = (jnp.broadcast_to(seg, (BB, RB))
                  == lax.broadcasted_iota(jnp.int32, (BB, RB), 0)
                  ).astype(jnp.float32)
        acc[...] += jnp.dot(onehot, h, preferred_element_type=jnp.float32)
        cnt[...] += jnp.broadcast_to(
            jnp.sum(onehot, axis=1, keepdims=True), (BB, DD))

        @pl.when(i == NRB - 1)
        def _():
            o_r[...] = acc[...] / jnp.maximum(cnt[...], 1.0)

    row = pl.BlockSpec((RB, DD), lambda i: (i, 0))
    row_hi = pl.BlockSpec((RB, DD), lambda i: (i + NRB, 0))
    full = pl.BlockSpec((DD, DD), lambda i: (0, 0))
    bias = pl.BlockSpec((1, DD), lambda i: (0, 0))
    return pl.pallas_call(
        body,
        grid=(NRB,),
        in_specs=[row, row, row_hi, full, bias, full, bias,
                  pl.BlockSpec((1, 1, RB), lambda i: (i, 0, 0))],
        out_specs=pl.BlockSpec((BB, DD), lambda i: (0, 0)),
        out_shape=jax.ShapeDtypeStruct((BB, DD), jnp.float32),
        scratch_shapes=[pltpu.VMEM((BB, DD), jnp.float32),
                        pltpu.VMEM((BB, DD), jnp.float32)],
    )(h0, aggs, aggs, W1, b1.reshape(1, DD), W2, b2.reshape(1, DD), batch3d)


def kernel(x, edge_index, batch, W1_0, b1_0, W2_0, b2_0, W1_1, b1_1, W2_1, b2_1):
    ei4 = edge_index.reshape(2, NTILES, NCHUNK, CH).transpose(1, 2, 0, 3)
    x_pad = jnp.pad(x, ((0, NPAD - NN), (0, 0)))
    batch3d = jnp.pad(batch, (0, NPAD - NN),
                      constant_values=BB).reshape(NRB, 1, RB)

    aggs = _sc_agg(x_pad, ei4)
    h0 = _mlp0(x_pad, aggs, W1_0, b1_0, W2_0, b2_0)
    aggs1 = _sc_agg(h0, ei4)
    return _mlp1_pool(h0, aggs1, W1_1, b1_1, W2_1, b2_1, batch3d)


# speedup vs baseline: 1.0237x; 1.0163x over previous
import functools

import jax
import jax.numpy as jnp
from jax import lax
from jax.experimental import pallas as pl
from jax.experimental.pallas import tpu as pltpu
from jax.experimental.pallas import tpu_sc as plsc

NN = 10000
NPAD = 10112
EE = 320000
DD = 128
BB = 64
NTILES = 32
E_PER_TILE = EE // NTILES
CH = 80
NCHUNK = E_PER_TILE // CH
NBUF = 4
NIDX = 8
ROWS_PER_TILE = NPAD // 16
ZCH = ROWS_PER_TILE // 8
RB = 1264
NRB = NPAD // RB


def _sc_agg(table, ei2):
    mesh = plsc.VectorSubcoreMesh(core_axis_name="c", subcore_axis_name="s")
    nfull = table.shape[0] // ROWS_PER_TILE
    nrem = table.shape[0] % ROWS_PER_TILE

    @functools.partial(
        pl.kernel,
        out_type=jax.ShapeDtypeStruct((2 * NPAD, DD), jnp.float32),
        mesh=mesh,
        scratch_types=[
            pltpu.VMEM((NIDX, 2, CH), jnp.int32),
            [pltpu.VMEM((CH, DD), jnp.float32) for _ in range(NBUF)],
            pltpu.VMEM_SHARED((NPAD, DD), jnp.float32),
            [pltpu.SemaphoreType.DMA for _ in range(NIDX)],
            [pltpu.SemaphoreType.DMA for _ in range(NBUF)],
            [pltpu.SemaphoreType.DMA for _ in range(NBUF)],
        ],
    )
    def k(table_h, ei_h, out_h, ring, rows, acc_s, isem, gsem, ssem):
        cid = lax.axis_index("c")
        sid = lax.axis_index("s")
        tile = cid * 16 + sid

        def zrow(r, _):
            def zcol(j, _):
                rows[0][r, pl.ds(j * 16, 16)] = jnp.zeros((16,), jnp.float32)
                return 0
            return lax.fori_loop(0, DD // 16, zcol, 0)
        lax.fori_loop(0, CH, zrow, 0)

        def zfill(base, count):
            full, rem = count // ZCH, count % ZCH
            for i in range(full):
                pltpu.sync_copy(rows[0].at[pl.ds(0, ZCH)],
                                acc_s.at[pl.ds(base + i * ZCH, ZCH)])
            if rem:
                pltpu.sync_copy(rows[0].at[pl.ds(0, rem)],
                                acc_s.at[pl.ds(base + full * ZCH, rem)])

        base = sid * ROWS_PER_TILE

        @pl.when(jnp.logical_and(cid == 0, sid < nfull))
        def _():
            pltpu.sync_copy(table_h.at[pl.ds(base, ROWS_PER_TILE)],
                            acc_s.at[pl.ds(base, ROWS_PER_TILE)])
        if nrem:
            @pl.when(jnp.logical_and(cid == 0, sid == nfull))
            def _():
                pltpu.sync_copy(table_h.at[pl.ds(base, nrem)],
                                acc_s.at[pl.ds(base, nrem)])
                zfill(base + nrem, ROWS_PER_TILE - nrem)

            @pl.when(jnp.logical_and(cid == 0, sid > nfull))
            def _():
                zfill(base, ROWS_PER_TILE)

        @pl.when(cid == 1)
        def _():
            zfill(base, ROWS_PER_TILE)
        plsc.subcore_barrier()

        def iload(c, s):
            pltpu.async_copy(ei_h.at[0, tile, c], ring.at[s, pl.ds(0, 1)],
                             isem[s])
            pltpu.async_copy(ei_h.at[1, tile, c], ring.at[s, pl.ds(1, 1)],
                             isem[s])

        def iwait(c, s):
            pltpu.make_async_copy(ei_h.at[0, tile, c],
                                  ring.at[s, pl.ds(0, 1)], isem[s]).wait()
            pltpu.make_async_copy(ei_h.at[1, tile, c],
                                  ring.at[s, pl.ds(1, 1)], isem[s]).wait()

        def gstart(s, b):
            pltpu.async_copy(table_h.at[ring.at[s, 0]], rows[b], gsem[b])

        def gwait(s, b):
            pltpu.make_async_copy(table_h.at[ring.at[s, 0]], rows[b],
                                  gsem[b]).wait()

        def sstart(s, b):
            pltpu.async_copy(rows[b], acc_s.at[ring.at[s, 1]], ssem[b],
                             add=True)

        def swait(s, b):
            pltpu.make_async_copy(rows[b], acc_s.at[ring.at[s, 1]],
                                  ssem[b]).wait()

        def iter_ops(j, jm8, jm4, do_swait, do_iload, do_gather):
            gwait(jm8, jm4)
            sstart(jm8, jm4)
            if do_swait:
                swait((jm8 + 7) % NIDX, (jm4 + 3) % NBUF)
            if do_iload:
                iload(j + 6, (jm8 + 6) % NIDX)
            if do_gather:
                iwait(j + 3, (jm8 + 3) % NIDX)
                gstart((jm8 + 3) % NIDX, (jm4 + 3) % NBUF)

        for c in range(6):
            iload(c, c)
        for c in range(3):
            iwait(c, c)
            gstart(c, c)
        iter_ops(0, 0, 0, False, True, True)
        for j in range(1, 10):
            iter_ops(j, j % NIDX, j % NBUF, True, True, True)

        def body(o, _):
            j0 = 10 + o * 8
            for t in range(8):
                iter_ops(j0 + t, (10 + t) % NIDX, (10 + t) % NBUF,
                         True, True, True)
            return 0
        lax.fori_loop(0, 13, body, 0)

        for j in range(114, 119):
            iter_ops(j, j % NIDX, j % NBUF, True, True, True)
        for j in range(119, 122):
            iter_ops(j, j % NIDX, j % NBUF, True, False, True)
        for j in (122, 123, 124):
            jm8, jm4 = j % NIDX, j % NBUF
            gwait(jm8, jm4)
            sstart(jm8, jm4)
            swait((jm8 + 7) % NIDX, (jm4 + 3) % NBUF)
        swait(124 % NIDX, 124 % NBUF)
        plsc.subcore_barrier()

        pltpu.sync_copy(
            acc_s.at[pl.ds(sid * ROWS_PER_TILE, ROWS_PER_TILE)],
            out_h.at[pl.ds(cid * NPAD + sid * ROWS_PER_TILE, ROWS_PER_TILE)])

    return k(table, ei2)


def _mlp0(aggs, W1, b1, W2, b2):
    def body(a0_r, a1_r, w1_r, b1_r, w2_r, b2_r, o_r):
        h = a0_r[...] + a1_r[...]
        h = jnp.maximum(
            jnp.dot(h, w1_r[...], preferred_element_type=jnp.float32)
            + b1_r[...], 0.0)
        h = jnp.dot(h, w2_r[...], preferred_element_type=jnp.float32) + b2_r[...]
        o_r[...] = jnp.maximum(h, 0.0)

    row = pl.BlockSpec((RB, DD), lambda i: (i, 0))
    row_hi = pl.BlockSpec((RB, DD), lambda i: (i + NRB, 0))
    full = pl.BlockSpec((DD, DD), lambda i: (0, 0))
    bias = pl.BlockSpec((1, DD), lambda i: (0, 0))
    return pl.pallas_call(
        body,
        grid=(NRB,),
        in_specs=[row, row_hi, full, bias, full, bias],
        out_specs=row,
        out_shape=jax.ShapeDtypeStruct((NPAD, DD), jnp.float32),
    )(aggs, aggs, W1, b1.reshape(1, DD), W2, b2.reshape(1, DD))


def _mlp1_pool(aggs, W1, b1, W2, b2, batch3d):
    def body(a0_r, a1_r, w1_r, b1_r, w2_r, b2_r, bt_r, o_r, acc, cnt):
        i = pl.program_id(0)

        @pl.when(i == 0)
        def _():
            acc[...] = jnp.zeros_like(acc)
            cnt[...] = jnp.zeros_like(cnt)

        h = a0_r[...] + a1_r[...]
        h = jnp.maximum(
            jnp.dot(h, w1_r[...], preferred_element_type=jnp.float32)
            + b1_r[...], 0.0)
        h = jnp.dot(h, w2_r[...], preferred_element_type=jnp.float32) + b2_r[...]

        seg = bt_r[...].reshape(1, RB)
        onehot = (jnp.broadcast_to(seg, (BB, RB))
                  == lax.broadcasted_iota(jnp.int32, (BB, RB), 0)
                  ).astype(jnp.float32)
        acc[...] += jnp.dot(onehot, h, preferred_element_type=jnp.float32)
        cnt[...] += jnp.broadcast_to(
            jnp.sum(onehot, axis=1, keepdims=True), (BB, DD))

        @pl.when(i == NRB - 1)
        def _():
            o_r[...] = acc[...] / jnp.maximum(cnt[...], 1.0)

    row = pl.BlockSpec((RB, DD), lambda i: (i, 0))
    row_hi = pl.BlockSpec((RB, DD), lambda i: (i + NRB, 0))
    full = pl.BlockSpec((DD, DD), lambda i: (0, 0))
    bias = pl.BlockSpec((1, DD), lambda i: (0, 0))
    return pl.pallas_call(
        body,
        grid=(NRB,),
        in_specs=[row, row_hi, full, bias, full, bias,
                  pl.BlockSpec((1, 1, RB), lambda i: (i, 0, 0))],
        out_specs=pl.BlockSpec((BB, DD), lambda i: (0, 0)),
        out_shape=jax.ShapeDtypeStruct((BB, DD), jnp.float32),
        scratch_shapes=[pltpu.VMEM((BB, DD), jnp.float32),
                        pltpu.VMEM((BB, DD), jnp.float32)],
    )(aggs, aggs, W1, b1.reshape(1, DD), W2, b2.reshape(1, DD), batch3d)


def kernel(x, edge_index, batch, W1_0, b1_0, W2_0, b2_0, W1_1, b1_1, W2_1, b2_1):
    ei2 = edge_index.reshape(2, NTILES, NCHUNK, 1, CH)
    batch3d = jnp.pad(batch, (0, NPAD - NN),
                      constant_values=BB).reshape(NRB, 1, RB)

    aggs = _sc_agg(x, ei2)
    h0 = _mlp0(aggs, W1_0, b1_0, W2_0, b2_0)
    aggs1 = _sc_agg(h0, ei2)
    return _mlp1_pool(aggs1, W1_1, b1_1, W2_1, b2_1, batch3d)


# speedup vs baseline: 1.0628x; 1.0382x over previous
import functools

import jax
import jax.numpy as jnp
from jax import lax
from jax.experimental import pallas as pl
from jax.experimental.pallas import tpu as pltpu
from jax.experimental.pallas import tpu_sc as plsc

NN = 10000
NPAD = 10112
EE = 320000
DD = 128
BB = 64
NTILES = 32
E_PER_TILE = EE // NTILES
CH = 80
NCHUNK = E_PER_TILE // CH
NBUF = 4
NIDX = 8
ROWS_PER_TILE = NPAD // 16
ZCH = ROWS_PER_TILE // 8
RB = 1264
NRB = NPAD // RB


def _sc_agg(table, ei2):
    mesh = plsc.VectorSubcoreMesh(core_axis_name="c", subcore_axis_name="s")
    nfull = table.shape[0] // ROWS_PER_TILE
    nrem = table.shape[0] % ROWS_PER_TILE

    @functools.partial(
        pl.kernel,
        out_type=jax.ShapeDtypeStruct((2 * NPAD, DD), jnp.float32),
        mesh=mesh,
        scratch_types=[
            pltpu.VMEM((NIDX, 2, CH), jnp.int32),
            [pltpu.VMEM((CH, DD), jnp.float32) for _ in range(NBUF)],
            pltpu.VMEM_SHARED((NPAD, DD), jnp.float32),
            [pltpu.SemaphoreType.DMA for _ in range(NIDX)],
            [pltpu.SemaphoreType.DMA for _ in range(NBUF)],
            [pltpu.SemaphoreType.DMA for _ in range(NBUF)],
        ],
    )
    def k(table_h, ei_h, out_h, ring, rows, acc_s, isem, gsem, ssem):
        cid = lax.axis_index("c")
        sid = lax.axis_index("s")
        tile = cid * 16 + sid

        def zrow(r, _):
            def zcol(j, _):
                rows[0][r, pl.ds(j * 16, 16)] = jnp.zeros((16,), jnp.float32)
                return 0
            return lax.fori_loop(0, DD // 16, zcol, 0)
        lax.fori_loop(0, CH, zrow, 0)

        def zfill(base, count):
            full, rem = count // ZCH, count % ZCH
            for i in range(full):
                pltpu.sync_copy(rows[0].at[pl.ds(0, ZCH)],
                                acc_s.at[pl.ds(base + i * ZCH, ZCH)])
            if rem:
                pltpu.sync_copy(rows[0].at[pl.ds(0, rem)],
                                acc_s.at[pl.ds(base + full * ZCH, rem)])

        base = sid * ROWS_PER_TILE

        @pl.when(jnp.logical_and(cid == 0, sid < nfull))
        def _():
            pltpu.sync_copy(table_h.at[pl.ds(base, ROWS_PER_TILE)],
                            acc_s.at[pl.ds(base, ROWS_PER_TILE)])
        if nrem:
            @pl.when(jnp.logical_and(cid == 0, sid == nfull))
            def _():
                pltpu.sync_copy(table_h.at[pl.ds(base, nrem)],
                                acc_s.at[pl.ds(base, nrem)])
                zfill(base + nrem, ROWS_PER_TILE - nrem)

            @pl.when(jnp.logical_and(cid == 0, sid > nfull))
            def _():
                zfill(base, ROWS_PER_TILE)

        @pl.when(cid == 1)
        def _():
            zfill(base, ROWS_PER_TILE)
        plsc.subcore_barrier()

        def iload(c, s):
            off = pl.multiple_of(tile * E_PER_TILE + c * CH, 8)
            pltpu.async_copy(ei_h.at[pl.ds(off, CH)], ring.at[s, 0], isem[s])
            off2 = pl.multiple_of(EE + tile * E_PER_TILE + c * CH, 8)
            pltpu.async_copy(ei_h.at[pl.ds(off2, CH)], ring.at[s, 1], isem[s])

        def iwait(c, s):
            off = pl.multiple_of(tile * E_PER_TILE + c * CH, 8)
            pltpu.make_async_copy(ei_h.at[pl.ds(off, CH)], ring.at[s, 0],
                                  isem[s]).wait()
            off2 = pl.multiple_of(EE + tile * E_PER_TILE + c * CH, 8)
            pltpu.make_async_copy(ei_h.at[pl.ds(off2, CH)], ring.at[s, 1],
                                  isem[s]).wait()

        def gstart(s, b):
            pltpu.async_copy(table_h.at[ring.at[s, 0]], rows[b], gsem[b])

        def gwait(s, b):
            pltpu.make_async_copy(table_h.at[ring.at[s, 0]], rows[b],
                                  gsem[b]).wait()

        def sstart(s, b):
            pltpu.async_copy(rows[b], acc_s.at[ring.at[s, 1]], ssem[b],
                             add=True)

        def swait(s, b):
            pltpu.make_async_copy(rows[b], acc_s.at[ring.at[s, 1]],
                                  ssem[b]).wait()

        def iter_ops(j, jm8, jm4, do_swait, do_iload, do_gather):
            gwait(jm8, jm4)
            sstart(jm8, jm4)
            if do_swait:
                swait((jm8 + 7) % NIDX, (jm4 + 3) % NBUF)
            if do_iload:
                iload(j + 6, (jm8 + 6) % NIDX)
            if do_gather:
                iwait(j + 3, (jm8 + 3) % NIDX)
                gstart((jm8 + 3) % NIDX, (jm4 + 3) % NBUF)

        for c in range(6):
            iload(c, c)
        for c in range(3):
            iwait(c, c)
            gstart(c, c)
        iter_ops(0, 0, 0, False, True, True)
        for j in range(1, 10):
            iter_ops(j, j % NIDX, j % NBUF, True, True, True)

        def body(o, _):
            j0 = 10 + o * 8
            for t in range(8):
                iter_ops(j0 + t, (10 + t) % NIDX, (10 + t) % NBUF,
                         True, True, True)
            return 0
        lax.fori_loop(0, 13, body, 0)

        for j in range(114, 119):
            iter_ops(j, j % NIDX, j % NBUF, True, True, True)
        for j in range(119, 122):
            iter_ops(j, j % NIDX, j % NBUF, True, False, True)
        for j in (122, 123, 124):
            jm8, jm4 = j % NIDX, j % NBUF
            gwait(jm8, jm4)
            sstart(jm8, jm4)
            swait((jm8 + 7) % NIDX, (jm4 + 3) % NBUF)
        swait(124 % NIDX, 124 % NBUF)
        plsc.subcore_barrier()

        pltpu.sync_copy(
            acc_s.at[pl.ds(sid * ROWS_PER_TILE, ROWS_PER_TILE)],
            out_h.at[pl.ds(cid * NPAD + sid * ROWS_PER_TILE, ROWS_PER_TILE)])

    return k(table, ei2)


def _mlp0(aggs, W1, b1, W2, b2):
    def body(a0_r, a1_r, w1_r, b1_r, w2_r, b2_r, o_r):
        h = a0_r[...] + a1_r[...]
        h = jnp.maximum(
            jnp.dot(h, w1_r[...], preferred_element_type=jnp.float32)
            + b1_r[...], 0.0)
        h = jnp.dot(h, w2_r[...], preferred_element_type=jnp.float32) + b2_r[...]
        o_r[...] = jnp.maximum(h, 0.0)

    row = pl.BlockSpec((RB, DD), lambda i: (i, 0))
    row_hi = pl.BlockSpec((RB, DD), lambda i: (i + NRB, 0))
    full = pl.BlockSpec((DD, DD), lambda i: (0, 0))
    bias = pl.BlockSpec((1, DD), lambda i: (0, 0))
    return pl.pallas_call(
        body,
        grid=(NRB,),
        in_specs=[row, row_hi, full, bias, full, bias],
        out_specs=row,
        out_shape=jax.ShapeDtypeStruct((NPAD, DD), jnp.float32),
    )(aggs, aggs, W1, b1.reshape(1, DD), W2, b2.reshape(1, DD))


def _mlp1_pool(aggs, W1, b1, W2, b2, batch3d):
    def body(a0_r, a1_r, w1_r, b1_r, w2_r, b2_r, bt_r, o_r, acc, cnt):
        i = pl.program_id(0)

        @pl.when(i == 0)
        def _():
            acc[...] = jnp.zeros_like(acc)
            cnt[...] = jnp.zeros_like(cnt)

        h = a0_r[...] + a1_r[...]
        h = jnp.maximum(
            jnp.dot(h, w1_r[...], preferred_element_type=jnp.float32)
            + b1_r[...], 0.0)
        h = jnp.dot(h, w2_r[...], preferred_element_type=jnp.float32) + b2_r[...]

        seg = bt_r[...].reshape(1, RB)
        onehot = (jnp.broadcast_to(seg, (BB, RB))
                  == lax.broadcasted_iota(jnp.int32, (BB, RB), 0)
                  ).astype(jnp.float32)
        acc[...] += jnp.dot(onehot, h, preferred_element_type=jnp.float32)
        cnt[...] += jnp.broadcast_to(
            jnp.sum(onehot, axis=1, keepdims=True), (BB, DD))

        @pl.when(i == NRB - 1)
        def _():
            o_r[...] = acc[...] / jnp.maximum(cnt[...], 1.0)

    row = pl.BlockSpec((RB, DD), lambda i: (i, 0))
    row_hi = pl.BlockSpec((RB, DD), lambda i: (i + NRB, 0))
    full = pl.BlockSpec((DD, DD), lambda i: (0, 0))
    bias = pl.BlockSpec((1, DD), lambda i: (0, 0))
    return pl.pallas_call(
        body,
        grid=(NRB,),
        in_specs=[row, row_hi, full, bias, full, bias,
                  pl.BlockSpec((1, 1, RB), lambda i: (i, 0, 0))],
        out_specs=pl.BlockSpec((BB, DD), lambda i: (0, 0)),
        out_shape=jax.ShapeDtypeStruct((BB, DD), jnp.float32),
        scratch_shapes=[pltpu.VMEM((BB, DD), jnp.float32),
                        pltpu.VMEM((BB, DD), jnp.float32)],
    )(aggs, aggs, W1, b1.reshape(1, DD), W2, b2.reshape(1, DD), batch3d)


def kernel(x, edge_index, batch, W1_0, b1_0, W2_0, b2_0, W1_1, b1_1, W2_1, b2_1):
    ei2 = edge_index.reshape(2 * EE)
    batch3d = jnp.pad(batch, (0, NPAD - NN),
                      constant_values=BB).reshape(NRB, 1, RB)

    aggs = _sc_agg(x, ei2)
    h0 = _mlp0(aggs, W1_0, b1_0, W2_0, b2_0)
    aggs1 = _sc_agg(h0, ei2)
    return _mlp1_pool(aggs1, W1_1, b1_1, W2_1, b2_1, batch3d)
